# baseline (speedup 1.0000x reference)
import jax
import jax.numpy as jnp
from jax.experimental import pallas as pl
from jax.experimental.pallas import tpu as pltpu

PRE_SEQ_LEN = 128
BATCH = 16
EMB_DIM = 18432
N_ROWS = BATCH * PRE_SEQ_LEN
SUB = 144
LANE = 128
R = 16
N_STEPS = N_ROWS // R


def _gather_body(idx_ref, tbl_ref, out_ref):
    step = pl.program_id(0)
    base = step * R
    for r in range(R):
        idx = idx_ref[base + r]
        out_ref[r] = tbl_ref[idx]


def kernel(prefix, embedding_table):
    flat_idx = prefix.reshape(N_ROWS)
    tbl = embedding_table.reshape(PRE_SEQ_LEN, SUB, LANE)

    grid_spec = pltpu.PrefetchScalarGridSpec(
        num_scalar_prefetch=1,
        grid=(N_STEPS,),
        in_specs=[
            pl.BlockSpec(
                (PRE_SEQ_LEN, SUB, LANE), lambda i, idx_ref: (0, 0, 0)
            )
        ],
        out_specs=pl.BlockSpec((R, SUB, LANE), lambda i, idx_ref: (i, 0, 0)),
    )

    out = pl.pallas_call(
        _gather_body,
        grid_spec=grid_spec,
        out_shape=jax.ShapeDtypeStruct((N_ROWS, SUB, LANE), jnp.float32),
        compiler_params=pltpu.CompilerParams(
            dimension_semantics=("arbitrary",),
        ),
    )(flat_idx, tbl)
    return out.reshape(BATCH, PRE_SEQ_LEN, EMB_DIM)


# speedup vs baseline: 1.1139x; 1.1139x over previous
import jax
import jax.numpy as jnp
from jax.experimental import pallas as pl
from jax.experimental.pallas import tpu as pltpu

PRE_SEQ_LEN = 128
BATCH = 16
EMB_DIM = 18432
N_ROWS = BATCH * PRE_SEQ_LEN
SUB = 144
LANE = 128
UNROLL = 8


def _gather_body(idx_ref, tbl_hbm, out_hbm, tbl_vmem, sem_t, sem):
    cp_t = pltpu.make_async_copy(tbl_hbm, tbl_vmem, sem_t)
    cp_t.start()
    cp_t.wait()

    def row_copy(k):
        idx = idx_ref[k]
        return pltpu.make_async_copy(
            tbl_vmem.at[pl.ds(idx, 1)],
            out_hbm.at[pl.ds(k, 1)],
            sem,
        )

    def issue(g, carry):
        for j in range(UNROLL):
            row_copy(g * UNROLL + j).start()
        return carry

    jax.lax.fori_loop(0, N_ROWS // UNROLL, issue, 0)

    def drain(g, carry):
        for j in range(UNROLL):
            row_copy(g * UNROLL + j).wait()
        return carry

    jax.lax.fori_loop(0, N_ROWS // UNROLL, drain, 0)


def kernel(prefix, embedding_table):
    flat_idx = prefix.reshape(N_ROWS)
    tbl = embedding_table.reshape(PRE_SEQ_LEN, SUB, LANE)

    grid_spec = pltpu.PrefetchScalarGridSpec(
        num_scalar_prefetch=1,
        grid=(1,),
        in_specs=[pl.BlockSpec(memory_space=pl.ANY)],
        out_specs=pl.BlockSpec(memory_space=pl.ANY),
        scratch_shapes=[
            pltpu.VMEM((PRE_SEQ_LEN, SUB, LANE), jnp.float32),
            pltpu.SemaphoreType.DMA,
            pltpu.SemaphoreType.DMA,
        ],
    )

    out = pl.pallas_call(
        _gather_body,
        grid_spec=grid_spec,
        out_shape=jax.ShapeDtypeStruct((N_ROWS, SUB, LANE), jnp.float32),
    )(flat_idx, tbl)
    return out.reshape(BATCH, PRE_SEQ_LEN, EMB_DIM)
